# baseline (speedup 1.0000x reference)
import functools

import jax
import jax.numpy as jnp
from jax import lax
from jax.experimental import pallas as pl
from jax.experimental.pallas import tpu as pltpu
from jax.experimental.pallas import tpu_sc as plsc

N_NODES = 10000
N_PAD = 10240
D = 128
N_EDGES = 320000
NC = 2
NS = 16
NW = NC * NS
CHUNK = 128
NCHUNKS = 80
E_PAD = NW * NCHUNKS * CHUNK
ROWS_PER_TILE = N_PAD // NS


def _sc_agg_body(x_hbm, ei_hbm, zero_hbm, out_hbm,
                 acc_sh, rows0, rows1, ib0, ib1, ib2, ib3,
                 gsem0, gsem1, isem0, isem1, isem2, isem3):
    c = lax.axis_index("c")
    s = lax.axis_index("s")
    r0 = s * ROWS_PER_TILE
    pltpu.sync_copy(zero_hbm.at[pl.ds(r0, ROWS_PER_TILE)],
                    acc_sh.at[pl.ds(r0, ROWS_PER_TILE)])
    wid = c * NS + s

    rows = (rows0, rows1)
    gsem = (gsem0, gsem1)
    ib = (ib0, ib1, ib2, ib3)
    isem = (isem0, isem1, isem2, isem3)

    pltpu.sync_copy(ei_hbm.at[wid, 0], ib0)
    pltpu.async_copy(ei_hbm.at[wid, 1], ib1, isem1)
    pltpu.async_copy(ei_hbm.at[wid, 2], ib2, isem2)
    plsc.subcore_barrier()
    pltpu.async_copy(x_hbm.at[ib0.at[0]], rows0, gsem0)

    def body(t, carry):
        for b in range(4):
            i = 4 * t + b
            rb, rbn = b % 2, (b + 1) % 2
            ibn, ibp = (b + 1) % 4, (b + 3) % 4
            pltpu.make_async_copy(x_hbm.at[ib0.at[0]], rows[rb],
                                  gsem[rb]).wait()
            pltpu.make_async_copy(ei_hbm.at[wid, 0], ib[ibn],
                                  isem[ibn]).wait()
            pltpu.async_copy(x_hbm.at[ib[ibn].at[0]], rows[rbn], gsem[rbn])
            pltpu.sync_copy(rows[rb], acc_sh.at[ib[b].at[1]], add=True)
            nx3 = jnp.minimum(i + 3, NCHUNKS - 1)
            pltpu.async_copy(ei_hbm.at[wid, nx3], ib[ibp], isem[ibp])
        return carry

    lax.fori_loop(0, NCHUNKS // 4, body, 0)
    pltpu.make_async_copy(x_hbm.at[ib0.at[0]], rows0, gsem0).wait()
    pltpu.make_async_copy(ei_hbm.at[wid, 0], ib1, isem1).wait()
    pltpu.make_async_copy(ei_hbm.at[wid, 0], ib2, isem2).wait()
    plsc.subcore_barrier()
    pltpu.sync_copy(acc_sh.at[pl.ds(r0, ROWS_PER_TILE)],
                    out_hbm.at[c, pl.ds(r0, ROWS_PER_TILE)])


_sc_agg = functools.partial(
    pl.kernel,
    mesh=plsc.VectorSubcoreMesh(core_axis_name="c", subcore_axis_name="s"),
    out_type=jax.ShapeDtypeStruct((NC, N_PAD, D), jnp.float32),
    scratch_types=[
        pltpu.VMEM_SHARED((N_PAD, D), jnp.float32),
        pltpu.VMEM((CHUNK, D), jnp.float32),
        pltpu.VMEM((CHUNK, D), jnp.float32),
        pltpu.VMEM((2, CHUNK), jnp.int32),
        pltpu.VMEM((2, CHUNK), jnp.int32),
        pltpu.VMEM((2, CHUNK), jnp.int32),
        pltpu.VMEM((2, CHUNK), jnp.int32),
        pltpu.SemaphoreType.DMA,
        pltpu.SemaphoreType.DMA,
        pltpu.SemaphoreType.DMA,
        pltpu.SemaphoreType.DMA,
        pltpu.SemaphoreType.DMA,
        pltpu.SemaphoreType.DMA,
    ],
)(_sc_agg_body)


BLK = 1024


def _tc_linear_body(p_ref, w_ref, b_ref, o_ref):
    agg = p_ref[0] + p_ref[1]
    o_ref[...] = lax.dot_general(
        agg, w_ref[...], (((1,), (1,)), ((), ())),
        preferred_element_type=jnp.float32) + b_ref[...]


def _tc_linear(partials, W, b):
    return pl.pallas_call(
        _tc_linear_body,
        grid=(N_PAD // BLK,),
        in_specs=[
            pl.BlockSpec((NC, BLK, D), lambda i: (0, i, 0)),
            pl.BlockSpec((D, D), lambda i: (0, 0)),
            pl.BlockSpec((1, D), lambda i: (0, 0)),
        ],
        out_specs=pl.BlockSpec((BLK, D), lambda i: (i, 0)),
        out_shape=jax.ShapeDtypeStruct((N_PAD, D), jnp.float32),
    )(partials, W, b.reshape(1, D))


def kernel(x, edge_index, W, b):
    src = edge_index[0].astype(jnp.int32)
    dst = edge_index[1].astype(jnp.int32)
    npad = E_PAD - N_EDGES
    src = jnp.concatenate([src, jnp.zeros((npad,), jnp.int32)])
    dst = jnp.concatenate(
        [dst, N_NODES + (jnp.arange(npad, dtype=jnp.int32) % (N_PAD - N_NODES))])
    src3 = src.reshape(NW, NCHUNKS, 1, CHUNK)
    dst3 = dst.reshape(NW, NCHUNKS, 1, CHUNK)
    ei4 = jnp.concatenate([src3, dst3], axis=2)
    zero = jnp.zeros((N_PAD, D), jnp.float32)
    partials = _sc_agg(x, ei4, zero)
    return _tc_linear(partials, W, b)[:N_NODES]
